# baseline (speedup 1.0000x reference)
import functools

import jax
import jax.numpy as jnp
from jax import lax
from jax.experimental import pallas as pl
from jax.experimental.pallas import tpu as pltpu
from jax.experimental.pallas import tpu_sc as plsc

_POOL = 7
_NPOS = _POOL * _POOL


def _unpack2(g):
    return plsc.unpack(plsc.bitcast(g, jnp.bfloat16),
                       format=plsc.PackFormat.INTERLEAVED,
                       preferred_element_type=jnp.float32)


def _roi_body(NSBM, NB, rois_hbm, table_hbm, img_hbm, out_hbm,
              table_v, rois_v, img_v, outst, sem):
    cidx = lax.axis_index("c")
    sidx = lax.axis_index("s")
    wid = sidx * 2 + cidx
    cg = lax.rem(wid, 4)
    rg = lax.div(wid, 4)

    pltpu.sync_copy(img_hbm, img_v)
    pltpu.sync_copy(table_hbm.at[cg], table_v)
    pltpu.sync_copy(rois_hbm.at[rg], rois_v)

    iv = img_v[:]
    sy63 = iv[0]
    syd = iv[1]
    sx63 = iv[2]
    sxd = iv[3]

    lane = lax.iota(jnp.int32, 16)
    r0 = rg * (NSBM * 16)
    nsb = jnp.minimum(NSBM, NB - rg * NSBM)
    cgo = cg * 16

    def out_dst(sb):
        return out_hbm.at[pl.ds(r0 + sb * 16, 16), :, pl.ds(cgo, 16)]

    def sb_body(sb, _):
        buf = lax.rem(sb, 2)
        buf_vec = jnp.full((16,), buf, dtype=jnp.int32)

        @pl.when(sb >= 2)
        def _():
            pltpu.make_async_copy(outst.at[buf], out_dst(sb - 2), sem).wait()

        y1 = rois_v[0, sb]
        x1 = rois_v[1, sb]
        y2 = rois_v[2, sb]
        x2 = rois_v[3, sb]
        ay = y1 * sy63
        dy = (y2 - y1) * syd
        ax = x1 * sx63
        dx = (x2 - x1) * sxd

        xs = []
        for j in range(_POOL):
            inx = ax + float(j) * dx
            vx = (inx >= 0.0) & (inx <= 63.0)
            cx = jnp.clip(inx, 0.0, 63.0)
            li = cx.astype(jnp.int32)
            lx = cx - li.astype(jnp.float32)
            mx = jnp.where(vx, 1.0, 0.0)
            xs.append((li, lx, mx))

        @plsc.parallel_loop(0, _POOL, unroll=2)
        def i_body(i):
            fi = lax.convert_element_type(i, jnp.float32)
            iny = ay + fi * dy
            vy = (iny >= 0.0) & (iny <= 63.0)
            cy = jnp.clip(iny, 0.0, 63.0)
            ti = cy.astype(jnp.int32)
            ly = cy - ti.astype(jnp.float32)
            my = jnp.where(vy, 1.0, 0.0)
            oy = 1.0 - ly
            bstep = jnp.where(ly > 0.0, 64, 0)
            tb = ti * 64
            for j in range(_POOL):
                li, lx, mx = xs[j]
                m = my * mx
                moy = m * oy
                mly = m * ly
                olx = 1.0 - lx
                w00 = moy * olx
                w01 = moy * lx
                w10 = mly * olx
                w11 = mly * lx
                tl = tb + li
                rstep = jnp.where(lx > 0.0, 1, 0)
                tr = tl + rstep
                bl = tl + bstep
                br = bl + rstep
                p_vec = jnp.full((16,), i * _POOL + j, dtype=jnp.int32)
                for w in range(8):
                    wb = w * 4096
                    a_tl, b_tl = _unpack2(plsc.load_gather(table_v, [tl + wb]))
                    a_tr, b_tr = _unpack2(plsc.load_gather(table_v, [tr + wb]))
                    a_bl, b_bl = _unpack2(plsc.load_gather(table_v, [bl + wb]))
                    a_br, b_br = _unpack2(plsc.load_gather(table_v, [br + wb]))
                    ve = w00 * a_tl + w01 * a_tr + w10 * a_bl + w11 * a_br
                    vo = w00 * b_tl + w01 * b_tr + w10 * b_bl + w11 * b_br
                    ce = jnp.full((16,), 2 * w, dtype=jnp.int32)
                    co = jnp.full((16,), 2 * w + 1, dtype=jnp.int32)
                    plsc.store_scatter(outst, [buf_vec, lane, p_vec, ce], ve)
                    plsc.store_scatter(outst, [buf_vec, lane, p_vec, co], vo)

        pltpu.async_copy(outst.at[buf], out_dst(sb), sem)
        return 0

    lax.fori_loop(0, nsb, sb_body, 0)
    pltpu.make_async_copy(outst.at[0], out_dst(nsb - 2), sem).wait()
    pltpu.make_async_copy(outst.at[1], out_dst(nsb - 1), sem).wait()


def kernel(feature, rois, img_size):
    N = rois.shape[0]
    assert N % 16 == 0
    H, W, C = feature.shape[1], feature.shape[2], feature.shape[3]
    NB = N // 16
    NSBM = -(-NB // 8)
    NPAD = NSBM * 8 * 16

    fb = feature[0].astype(jnp.bfloat16).reshape(H * W, 4, 8, 2)
    table4 = (lax.bitcast_convert_type(fb, jnp.int32)
              .transpose(1, 2, 0).reshape(4, H * W * 8))
    rois_p = jnp.pad(rois.astype(jnp.float32), ((0, NPAD - N), (0, 0)))
    rois4 = rois_p.T.reshape(4, 8, NSBM, 16).transpose(1, 0, 2, 3)
    hs = img_size[0].astype(jnp.float32)
    ws = img_size[1].astype(jnp.float32)
    imgp = jnp.stack([63.0 / hs, 10.5 / hs, 63.0 / ws, 10.5 / ws])
    imgp = jnp.concatenate([imgp, jnp.zeros((12,), jnp.float32)])

    mesh = plsc.VectorSubcoreMesh(core_axis_name="c", subcore_axis_name="s")
    fn = pl.kernel(
        functools.partial(_roi_body, NSBM, NB),
        out_type=jax.ShapeDtypeStruct((N, _NPOS, C), jnp.float32),
        mesh=mesh,
        scratch_types=[
            pltpu.VMEM((H * W * 8,), jnp.int32),
            pltpu.VMEM((4, NSBM, 16), jnp.float32),
            pltpu.VMEM((16,), jnp.float32),
            pltpu.VMEM((2, 16, _NPOS, 16), jnp.float32),
            pltpu.SemaphoreType.DMA,
        ],
        compiler_params=pltpu.CompilerParams(use_tc_tiling_on_sc=False,
                                             needs_layout_passes=False),
    )
    return fn(rois4, table4, imgp).reshape(N, _POOL, _POOL, C)


# speedup vs baseline: 1.5087x; 1.5087x over previous
import functools

import jax
import jax.numpy as jnp
from jax import lax
from jax.experimental import pallas as pl
from jax.experimental.pallas import tpu as pltpu
from jax.experimental.pallas import tpu_sc as plsc

_POOL = 7
_NPOS = _POOL * _POOL


def _unpack2(g):
    return plsc.unpack(plsc.bitcast(g, jnp.bfloat16),
                       format=plsc.PackFormat.INTERLEAVED,
                       preferred_element_type=jnp.float32)


def _roi_body(NSBM, NB, rois_hbm, table_hbm, img_hbm, out_hbm,
              table_v, rois_v, img_v, outst, sem):
    cidx = lax.axis_index("c")
    sidx = lax.axis_index("s")
    wid = sidx * 2 + cidx
    cg = lax.rem(wid, 4)
    rg = lax.div(wid, 4)

    pltpu.sync_copy(img_hbm, img_v)
    pltpu.sync_copy(table_hbm.at[cg], table_v)
    pltpu.sync_copy(rois_hbm.at[rg], rois_v)

    iv = img_v[:]
    sy63 = iv[0]
    syd = iv[1]
    sx63 = iv[2]
    sxd = iv[3]

    lane = lax.iota(jnp.int32, 16)
    r0 = rg * (NSBM * 16)
    nsb = jnp.minimum(NSBM, NB - rg * NSBM)
    cgo = cg * 16

    def out_dst(sb):
        return out_hbm.at[pl.ds(r0 + sb * 16, 16), :, pl.ds(cgo, 16)]

    def sb_body(sb, _):
        buf = lax.rem(sb, 2)
        buf_vec = jnp.full((16,), buf, dtype=jnp.int32)

        @pl.when(sb >= 2)
        def _():
            pltpu.make_async_copy(outst.at[buf], out_dst(sb - 2), sem).wait()

        y1 = rois_v[0, sb]
        x1 = rois_v[1, sb]
        y2 = rois_v[2, sb]
        x2 = rois_v[3, sb]
        ay = y1 * sy63
        dy = (y2 - y1) * syd
        ax = x1 * sx63
        dx = (x2 - x1) * sxd

        xs = []
        for j in range(_POOL):
            inx = ax + float(j) * dx
            vx = (inx >= 0.0) & (inx <= 63.0)
            cx = jnp.clip(inx, 0.0, 63.0)
            li = cx.astype(jnp.int32)
            lx = cx - li.astype(jnp.float32)
            mx = jnp.where(vx, 1.0, 0.0)
            xs.append((li, lx, mx))

        def i_body(i, _):
            fi = lax.convert_element_type(i, jnp.float32)
            iny = ay + fi * dy
            vy = (iny >= 0.0) & (iny <= 63.0)
            cy = jnp.clip(iny, 0.0, 63.0)
            ti = cy.astype(jnp.int32)
            ly = cy - ti.astype(jnp.float32)
            my = jnp.where(vy, 1.0, 0.0)
            oy = 1.0 - ly
            bstep = jnp.where(ly > 0.0, 64, 0)
            tb = ti * 64
            for j in range(_POOL):
                li, lx, mx = xs[j]
                m = my * mx
                moy = m * oy
                mly = m * ly
                olx = 1.0 - lx
                w00 = moy * olx
                w01 = moy * lx
                w10 = mly * olx
                w11 = mly * lx
                tl = tb + li
                rstep = jnp.where(lx > 0.0, 1, 0)
                tr = tl + rstep
                bl = tl + bstep
                br = bl + rstep
                p_vec = jnp.full((16,), i * _POOL + j, dtype=jnp.int32)
                for w in range(8):
                    wb = w * 4096
                    a_tl, b_tl = _unpack2(plsc.load_gather(table_v, [tl + wb]))
                    a_tr, b_tr = _unpack2(plsc.load_gather(table_v, [tr + wb]))
                    a_bl, b_bl = _unpack2(plsc.load_gather(table_v, [bl + wb]))
                    a_br, b_br = _unpack2(plsc.load_gather(table_v, [br + wb]))
                    ve = w00 * a_tl + w01 * a_tr + w10 * a_bl + w11 * a_br
                    vo = w00 * b_tl + w01 * b_tr + w10 * b_bl + w11 * b_br
                    ce = jnp.full((16,), 2 * w, dtype=jnp.int32)
                    co = jnp.full((16,), 2 * w + 1, dtype=jnp.int32)
                    plsc.store_scatter(outst, [buf_vec, lane, p_vec, ce], ve)
                    plsc.store_scatter(outst, [buf_vec, lane, p_vec, co], vo)
            return 0

        lax.fori_loop(0, _POOL, i_body, 0)
        pltpu.async_copy(outst.at[buf], out_dst(sb), sem)
        return 0

    lax.fori_loop(0, nsb, sb_body, 0)
    pltpu.make_async_copy(outst.at[0], out_dst(nsb - 2), sem).wait()
    pltpu.make_async_copy(outst.at[1], out_dst(nsb - 1), sem).wait()


def kernel(feature, rois, img_size):
    N = rois.shape[0]
    assert N % 16 == 0
    H, W, C = feature.shape[1], feature.shape[2], feature.shape[3]
    NB = N // 16
    NSBM = -(-NB // 8)
    NPAD = NSBM * 8 * 16

    fb = feature[0].astype(jnp.bfloat16).reshape(H * W, 4, 8, 2)
    table4 = (lax.bitcast_convert_type(fb, jnp.int32)
              .transpose(1, 2, 0).reshape(4, H * W * 8))
    rois_p = jnp.pad(rois.astype(jnp.float32), ((0, NPAD - N), (0, 0)))
    rois4 = rois_p.T.reshape(4, 8, NSBM, 16).transpose(1, 0, 2, 3)
    hs = img_size[0].astype(jnp.float32)
    ws = img_size[1].astype(jnp.float32)
    imgp = jnp.stack([63.0 / hs, 10.5 / hs, 63.0 / ws, 10.5 / ws])
    imgp = jnp.concatenate([imgp, jnp.zeros((12,), jnp.float32)])

    mesh = plsc.VectorSubcoreMesh(core_axis_name="c", subcore_axis_name="s")
    fn = pl.kernel(
        functools.partial(_roi_body, NSBM, NB),
        out_type=jax.ShapeDtypeStruct((N, _NPOS, C), jnp.float32),
        mesh=mesh,
        scratch_types=[
            pltpu.VMEM((H * W * 8,), jnp.int32),
            pltpu.VMEM((4, NSBM, 16), jnp.float32),
            pltpu.VMEM((16,), jnp.float32),
            pltpu.VMEM((2, 16, _NPOS, 16), jnp.float32),
            pltpu.SemaphoreType.DMA,
        ],
        compiler_params=pltpu.CompilerParams(use_tc_tiling_on_sc=False,
                                             needs_layout_passes=False),
    )
    return fn(rois4, table4, imgp).reshape(N, _POOL, _POOL, C)
